# Initial kernel scaffold; baseline (speedup 1.0000x reference)
import jax
import jax.numpy as jnp
from jax.experimental import pallas as pl


def kernel(embeddings, edge_index, in_proj_w, in_proj_b, out_proj_w, out_proj_b, lin_w, lin_b, ln_w, ln_b):
    raise NotImplementedError("write your pallas kernel here")


# speedup vs baseline: 2.7769x; 2.7769x over previous
import functools

import jax
import jax.numpy as jnp
from jax import lax
from jax.experimental import pallas as pl
from jax.experimental.pallas import tpu as pltpu
from jax.experimental.pallas import tpu_sc as plsc

N = 10000
NE = 320000
D = 128
H = 8
DH = D // H
SCALE = 1.0 / (DH ** 0.5)

N_PAD = 10240
FLAT = N_PAD * N_PAD
BC = 256
BN = 512
NI = N_PAD // BC
NJ = N_PAD // BN

NC = 2
NS = 16
HALF = N_PAD // NC
E_PER_SUB = NE // NS
EBATCH = 2000
NBATCH = E_PER_SUB // EBATCH
VPB = EBATCH // 16
CHUNKS = 320
ZCHUNK = 16384
PER_TILE = HALF * N_PAD // NS
NZ = PER_TILE // ZCHUNK


def _sc_mask_body(src_hbm, dst_hbm, mask_hbm, idx2d, val2d, zbuf, sbuf, dbuf,
                  zsem, ssem):
    cid = lax.axis_index("c")
    sid = lax.axis_index("s")
    lo = cid * HALF
    dumpbase = lo * N_PAD + N

    def zf(t, c):
        zbuf[pl.ds(t * 16, 16)] = jnp.zeros((16,), jnp.float32)
        return c
    lax.fori_loop(0, ZCHUNK // 16, zf, 0)

    zbase = cid * (HALF * N_PAD) + sid * PER_TILE

    def zi(t, c):
        pltpu.async_copy(zbuf, mask_hbm.at[pl.ds(zbase + t * ZCHUNK, ZCHUNK)],
                         zsem)
        return c
    lax.fori_loop(0, NZ, zi, 0)

    lanes = lax.iota(jnp.int32, 16)
    for r in range(312, CHUNKS):
        for g in range(8):
            idx2d[r, pl.ds(g * 16, 16)] = dumpbase + g * 16 + lanes
            val2d[r, pl.ds(g * 16, 16)] = jnp.zeros((16,), jnp.float32)

    ebase = sid * E_PER_SUB

    def batch_body(b, c):
        pltpu.sync_copy(src_hbm.at[pl.ds(ebase + b * EBATCH, EBATCH)], sbuf)
        pltpu.sync_copy(dst_hbm.at[pl.ds(ebase + b * EBATCH, EBATCH)], dbuf)

        def vec_body(i, cc):
            e = b * VPB + i
            r = e // 4
            col = (e % 4) * 32
            s16 = sbuf[pl.ds(i * 16, 16)]
            d16 = dbuf[pl.ds(i * 16, 16)]
            nsl = s16 != d16
            ok1 = (s16 >= lo) & (s16 < lo + HALF) & nsl
            ok2 = (d16 >= lo) & (d16 < lo + HALF) & nsl
            f1 = s16 * N_PAD + d16
            f2 = d16 * N_PAD + s16
            one = jnp.ones((16,), jnp.float32)
            zero = jnp.zeros((16,), jnp.float32)
            idx2d[r, pl.ds(col, 16)] = jnp.where(ok1, f1, dumpbase + col + lanes)
            val2d[r, pl.ds(col, 16)] = jnp.where(ok1, one, zero)
            idx2d[r, pl.ds(col + 16, 16)] = jnp.where(
                ok2, f2, dumpbase + col + 16 + lanes)
            val2d[r, pl.ds(col + 16, 16)] = jnp.where(ok2, one, zero)
            return cc
        lax.fori_loop(0, VPB, vec_body, 0)
        return c
    lax.fori_loop(0, NBATCH, batch_body, 0)

    def zd(t, c):
        pltpu.make_async_copy(zbuf, mask_hbm.at[pl.ds(zbase, ZCHUNK)],
                              zsem).wait()
        return c
    lax.fori_loop(0, NZ, zd, 0)
    plsc.subcore_barrier()

    def sc_outer(t, c):
        descs = []
        for kk in range(16):
            descs.append(pltpu.async_copy(
                val2d.at[t * 16 + kk], mask_hbm.at[idx2d.at[t * 16 + kk]],
                ssem))
        for dsc in descs:
            dsc.wait()
        return c
    lax.fori_loop(0, CHUNKS // 16, sc_outer, 0)


_sc_build_mask = functools.partial(
    pl.kernel,
    out_type=jax.ShapeDtypeStruct((FLAT,), jnp.float32),
    mesh=plsc.VectorSubcoreMesh(core_axis_name="c", subcore_axis_name="s"),
    scratch_types=[
        pltpu.VMEM((CHUNKS, 128), jnp.int32),
        pltpu.VMEM((CHUNKS, 128), jnp.float32),
        pltpu.VMEM((ZCHUNK,), jnp.float32),
        pltpu.VMEM((EBATCH,), jnp.int32),
        pltpu.VMEM((EBATCH,), jnp.int32),
        pltpu.SemaphoreType.DMA,
        pltpu.SemaphoreType.DMA,
    ],
)(_sc_mask_body)


def _proj_body(emb_ref, w_ref, b_ref, q_ref, k_ref, v_ref):
    y = lax.dot_general(emb_ref[...], w_ref[...], (((1,), (1,)), ((), ())),
                        preferred_element_type=jnp.float32) + b_ref[...]
    q_ref[...] = y[:, :D]
    k_ref[...] = y[:, D:2 * D]
    v_ref[...] = y[:, 2 * D:]


def _proj(emb_pad, w_all, b_all):
    bp = 512
    return pl.pallas_call(
        _proj_body,
        grid=(N_PAD // bp,),
        in_specs=[
            pl.BlockSpec((bp, D), lambda i: (i, 0)),
            pl.BlockSpec((3 * D, D), lambda i: (0, 0)),
            pl.BlockSpec((1, 3 * D), lambda i: (0, 0)),
        ],
        out_specs=[
            pl.BlockSpec((bp, D), lambda i: (i, 0)),
            pl.BlockSpec((bp, D), lambda i: (i, 0)),
            pl.BlockSpec((bp, D), lambda i: (i, 0)),
        ],
        out_shape=[jax.ShapeDtypeStruct((N_PAD, D), jnp.float32)] * 3,
    )(emb_pad, w_all, b_all)


def _flash_body(q_ref, k_ref, v_ref, mask_ref, emb_ref, wo_ref, bo_ref,
                wl_ref, bl_ref, lnw_ref, lnb_ref, out_ref,
                acc, mscr, lscr, cscr):
    j = pl.program_id(1)

    @pl.when(j == 0)
    def _():
        acc[...] = jnp.zeros((BC, D), jnp.float32)
        mscr[...] = jnp.full((BC, H), -jnp.inf, jnp.float32)
        lscr[...] = jnp.zeros((BC, H), jnp.float32)
        cscr[...] = jnp.zeros((BC, H), jnp.float32)

    mask = mask_ref[...]
    mb = mask > 0.0
    cscr[:, 0:1] = cscr[:, 0:1] + jnp.sum(mask, axis=1, keepdims=True)
    neg = -jnp.inf
    for h in range(H):
        qh = q_ref[:, h * DH:(h + 1) * DH]
        kh = k_ref[pl.ds(j * BN, BN), h * DH:(h + 1) * DH]
        s = lax.dot_general(qh, kh, (((1,), (1,)), ((), ())),
                            preferred_element_type=jnp.float32) * SCALE
        mo = mscr[:, h:h + 1]
        mc = jnp.max(jnp.where(mb, s, neg), axis=1, keepdims=True)
        mn = jnp.maximum(mo, mc)
        p = jnp.where(mb, jnp.exp(s - mn), 0.0)
        alpha = jnp.where(mn > neg, jnp.exp(mo - mn), 0.0)
        vh = v_ref[pl.ds(j * BN, BN), h * DH:(h + 1) * DH]
        pv = lax.dot_general(p, vh, (((1,), (0,)), ((), ())),
                             preferred_element_type=jnp.float32)
        acc[:, h * DH:(h + 1) * DH] = acc[:, h * DH:(h + 1) * DH] * alpha + pv
        lscr[:, h:h + 1] = lscr[:, h:h + 1] * alpha + jnp.sum(
            p, axis=1, keepdims=True)
        mscr[:, h:h + 1] = mn

    @pl.when(j == NJ - 1)
    def _():
        parts = [acc[:, h * DH:(h + 1) * DH] / lscr[:, h:h + 1]
                 for h in range(H)]
        ctx = jnp.concatenate(parts, axis=1)
        ctxp = lax.dot_general(ctx, wo_ref[...], (((1,), (1,)), ((), ())),
                               preferred_element_type=jnp.float32) + bo_ref[...]
        has = cscr[:, 0:1] > 0.0
        c2 = jnp.where(has, ctxp, emb_ref[...])
        h1 = lax.dot_general(c2, wl_ref[...], (((1,), (1,)), ((), ())),
                             preferred_element_type=jnp.float32) + bl_ref[...]
        mu = jnp.mean(h1, axis=1, keepdims=True)
        var = jnp.mean((h1 - mu) ** 2, axis=1, keepdims=True)
        hn = (h1 - mu) / jnp.sqrt(var + 1e-5) * lnw_ref[...] + lnb_ref[...]
        out_ref[...] = 0.5 * hn * (1.0 + lax.erf(hn * (2.0 ** -0.5)))


def _flash(q, k, v, mask2d, emb_pad, wo, bo, wl, bl, lnw, lnb):
    return pl.pallas_call(
        _flash_body,
        grid=(NI, NJ),
        in_specs=[
            pl.BlockSpec((BC, D), lambda i, j: (i, 0)),
            pl.BlockSpec((N_PAD, D), lambda i, j: (0, 0)),
            pl.BlockSpec((N_PAD, D), lambda i, j: (0, 0)),
            pl.BlockSpec((BC, BN), lambda i, j: (i, j)),
            pl.BlockSpec((BC, D), lambda i, j: (i, 0)),
            pl.BlockSpec((D, D), lambda i, j: (0, 0)),
            pl.BlockSpec((1, D), lambda i, j: (0, 0)),
            pl.BlockSpec((D, D), lambda i, j: (0, 0)),
            pl.BlockSpec((1, D), lambda i, j: (0, 0)),
            pl.BlockSpec((1, D), lambda i, j: (0, 0)),
            pl.BlockSpec((1, D), lambda i, j: (0, 0)),
        ],
        out_specs=pl.BlockSpec((BC, D), lambda i, j: (i, 0)),
        out_shape=jax.ShapeDtypeStruct((N_PAD, D), jnp.float32),
        scratch_shapes=[
            pltpu.VMEM((BC, D), jnp.float32),
            pltpu.VMEM((BC, H), jnp.float32),
            pltpu.VMEM((BC, H), jnp.float32),
            pltpu.VMEM((BC, H), jnp.float32),
        ],
        compiler_params=pltpu.CompilerParams(
            dimension_semantics=("arbitrary", "arbitrary")),
    )(q, k, v, mask2d, emb_pad, wo, bo, wl, bl, lnw, lnb)


def kernel(embeddings, edge_index, in_proj_w, in_proj_b, out_proj_w,
           out_proj_b, lin_w, lin_b, ln_w, ln_b):
    emb_pad = jnp.zeros((N_PAD, D), jnp.float32).at[:N].set(embeddings)
    src = edge_index[0].astype(jnp.int32)
    dst = edge_index[1].astype(jnp.int32)

    mask_flat = _sc_build_mask(src, dst)
    mask2d = mask_flat.reshape(N_PAD, N_PAD)

    q, k, v = _proj(emb_pad, in_proj_w, in_proj_b.reshape(1, 3 * D))
    out = _flash(q, k, v, mask2d, emb_pad,
                 out_proj_w, out_proj_b.reshape(1, D),
                 lin_w, lin_b.reshape(1, D),
                 ln_w.reshape(1, D), ln_b.reshape(1, D))
    return out[:N]


# speedup vs baseline: 17.7959x; 6.4085x over previous
import functools

import jax
import jax.numpy as jnp
from jax import lax
from jax.experimental import pallas as pl
from jax.experimental.pallas import tpu as pltpu
from jax.experimental.pallas import tpu_sc as plsc

N = 10000
NE = 320000
D = 128
H = 8
DH = D // H
SCALE = 1.0 / (DH ** 0.5)

N_PAD = 10240
FLAT = N_PAD * N_PAD
BC = 256
BN = 512
NI = N_PAD // BC
NJ = N_PAD // BN

NC = 2
NS = 16
NW = NC * NS
E_PER_W = NE // NW
EBATCH = 2000
NBATCH = E_PER_W // EBATCH
VPB = EBATCH // 16
CHUNKS = 160


def _sc_scatter_body(src_hbm, dst_hbm, mask_hbm, idx2d, val2d, sbuf, dbuf,
                     ssem):
    cid = lax.axis_index("c")
    sid = lax.axis_index("s")
    wid = cid * NS + sid
    dumpbase = N

    lanes = lax.iota(jnp.int32, 16)
    for r in range(156, CHUNKS):
        for g in range(8):
            idx2d[r, pl.ds(g * 16, 16)] = dumpbase + g * 16 + lanes
            val2d[r, pl.ds(g * 16, 16)] = jnp.zeros((16,), jnp.float32)

    ebase = wid * E_PER_W

    def batch_body(b, c):
        pltpu.sync_copy(src_hbm.at[pl.ds(ebase + b * EBATCH, EBATCH)], sbuf)
        pltpu.sync_copy(dst_hbm.at[pl.ds(ebase + b * EBATCH, EBATCH)], dbuf)

        def vec_body(i, cc):
            e = b * VPB + i
            r = e // 4
            col = (e % 4) * 32
            s16 = sbuf[pl.ds(i * 16, 16)]
            d16 = dbuf[pl.ds(i * 16, 16)]
            nsl = s16 != d16
            f1 = s16 * N_PAD + d16
            f2 = d16 * N_PAD + s16
            one = jnp.ones((16,), jnp.float32)
            zero = jnp.zeros((16,), jnp.float32)
            idx2d[r, pl.ds(col, 16)] = jnp.where(nsl, f1, dumpbase + col + lanes)
            val2d[r, pl.ds(col, 16)] = jnp.where(nsl, one, zero)
            idx2d[r, pl.ds(col + 16, 16)] = jnp.where(
                nsl, f2, dumpbase + col + 16 + lanes)
            val2d[r, pl.ds(col + 16, 16)] = jnp.where(nsl, one, zero)
            return cc
        lax.fori_loop(0, VPB, vec_body, 0)
        return c
    lax.fori_loop(0, NBATCH, batch_body, 0)

    def sc_outer(t, c):
        descs = []
        for kk in range(16):
            descs.append(pltpu.async_copy(
                val2d.at[t * 16 + kk], mask_hbm.at[idx2d.at[t * 16 + kk]],
                ssem))
        for dsc in descs:
            dsc.wait()
        return c
    lax.fori_loop(0, CHUNKS // 16, sc_outer, 0)


_sc_scatter = pl.kernel(
    _sc_scatter_body,
    out_type=(),
    mesh=plsc.VectorSubcoreMesh(core_axis_name="c", subcore_axis_name="s"),
    scratch_types=[
        pltpu.VMEM((CHUNKS, 128), jnp.int32),
        pltpu.VMEM((CHUNKS, 128), jnp.float32),
        pltpu.VMEM((EBATCH,), jnp.int32),
        pltpu.VMEM((EBATCH,), jnp.int32),
        pltpu.SemaphoreType.DMA,
    ],
)


def _proj_body(emb_ref, w_ref, b_ref, q_ref, k_ref, v_ref):
    y = lax.dot_general(emb_ref[...], w_ref[...], (((1,), (1,)), ((), ())),
                        preferred_element_type=jnp.float32) + b_ref[...]
    q_ref[...] = y[:, :D]
    k_ref[...] = y[:, D:2 * D]
    v_ref[...] = y[:, 2 * D:]


def _proj(emb_pad, w_all, b_all):
    bp = 512
    return pl.pallas_call(
        _proj_body,
        grid=(N_PAD // bp,),
        in_specs=[
            pl.BlockSpec((bp, D), lambda i: (i, 0)),
            pl.BlockSpec((3 * D, D), lambda i: (0, 0)),
            pl.BlockSpec((1, 3 * D), lambda i: (0, 0)),
        ],
        out_specs=[
            pl.BlockSpec((bp, D), lambda i: (i, 0)),
            pl.BlockSpec((bp, D), lambda i: (i, 0)),
            pl.BlockSpec((bp, D), lambda i: (i, 0)),
        ],
        out_shape=[jax.ShapeDtypeStruct((N_PAD, D), jnp.float32)] * 3,
    )(emb_pad, w_all, b_all)


def _flash_body(q_ref, k_ref, v_ref, mask_ref, emb_ref, wo_ref, bo_ref,
                wl_ref, bl_ref, lnw_ref, lnb_ref, out_ref,
                acc, mscr, lscr, cscr):
    j = pl.program_id(1)

    @pl.when(j == 0)
    def _():
        acc[...] = jnp.zeros((BC, D), jnp.float32)
        mscr[...] = jnp.full((BC, H), -jnp.inf, jnp.float32)
        lscr[...] = jnp.zeros((BC, H), jnp.float32)
        cscr[...] = jnp.zeros((BC, H), jnp.float32)

    mask = mask_ref[...]
    mb = mask > 0.0
    cscr[:, 0:1] = cscr[:, 0:1] + jnp.sum(mask, axis=1, keepdims=True)
    neg = -jnp.inf
    for h in range(H):
        qh = q_ref[:, h * DH:(h + 1) * DH]
        kh = k_ref[pl.ds(j * BN, BN), h * DH:(h + 1) * DH]
        s = lax.dot_general(qh, kh, (((1,), (1,)), ((), ())),
                            preferred_element_type=jnp.float32) * SCALE
        mo = mscr[:, h:h + 1]
        mc = jnp.max(jnp.where(mb, s, neg), axis=1, keepdims=True)
        mn = jnp.maximum(mo, mc)
        p = jnp.where(mb, jnp.exp(s - mn), 0.0)
        alpha = jnp.where(mn > neg, jnp.exp(mo - mn), 0.0)
        vh = v_ref[pl.ds(j * BN, BN), h * DH:(h + 1) * DH]
        pv = lax.dot_general(p, vh, (((1,), (0,)), ((), ())),
                             preferred_element_type=jnp.float32)
        acc[:, h * DH:(h + 1) * DH] = acc[:, h * DH:(h + 1) * DH] * alpha + pv
        lscr[:, h:h + 1] = lscr[:, h:h + 1] * alpha + jnp.sum(
            p, axis=1, keepdims=True)
        mscr[:, h:h + 1] = mn

    @pl.when(j == NJ - 1)
    def _():
        parts = [acc[:, h * DH:(h + 1) * DH] / lscr[:, h:h + 1]
                 for h in range(H)]
        ctx = jnp.concatenate(parts, axis=1)
        ctxp = lax.dot_general(ctx, wo_ref[...], (((1,), (1,)), ((), ())),
                               preferred_element_type=jnp.float32) + bo_ref[...]
        has = cscr[:, 0:1] > 0.0
        c2 = jnp.where(has, ctxp, emb_ref[...])
        h1 = lax.dot_general(c2, wl_ref[...], (((1,), (1,)), ((), ())),
                             preferred_element_type=jnp.float32) + bl_ref[...]
        mu = jnp.mean(h1, axis=1, keepdims=True)
        var = jnp.mean((h1 - mu) ** 2, axis=1, keepdims=True)
        hn = (h1 - mu) / jnp.sqrt(var + 1e-5) * lnw_ref[...] + lnb_ref[...]
        out_ref[...] = 0.5 * hn * (1.0 + lax.erf(hn * (2.0 ** -0.5)))


def _flash(q, k, v, mask2d, emb_pad, wo, bo, wl, bl, lnw, lnb):
    return pl.pallas_call(
        _flash_body,
        grid=(NI, NJ),
        in_specs=[
            pl.BlockSpec((BC, D), lambda i, j: (i, 0)),
            pl.BlockSpec((N_PAD, D), lambda i, j: (0, 0)),
            pl.BlockSpec((N_PAD, D), lambda i, j: (0, 0)),
            pl.BlockSpec((BC, BN), lambda i, j: (i, j)),
            pl.BlockSpec((BC, D), lambda i, j: (i, 0)),
            pl.BlockSpec((D, D), lambda i, j: (0, 0)),
            pl.BlockSpec((1, D), lambda i, j: (0, 0)),
            pl.BlockSpec((D, D), lambda i, j: (0, 0)),
            pl.BlockSpec((1, D), lambda i, j: (0, 0)),
            pl.BlockSpec((1, D), lambda i, j: (0, 0)),
            pl.BlockSpec((1, D), lambda i, j: (0, 0)),
        ],
        out_specs=pl.BlockSpec((BC, D), lambda i, j: (i, 0)),
        out_shape=jax.ShapeDtypeStruct((N_PAD, D), jnp.float32),
        scratch_shapes=[
            pltpu.VMEM((BC, D), jnp.float32),
            pltpu.VMEM((BC, H), jnp.float32),
            pltpu.VMEM((BC, H), jnp.float32),
            pltpu.VMEM((BC, H), jnp.float32),
        ],
        compiler_params=pltpu.CompilerParams(
            dimension_semantics=("arbitrary", "arbitrary")),
    )(q, k, v, mask2d, emb_pad, wo, bo, wl, bl, lnw, lnb)


def kernel(embeddings, edge_index, in_proj_w, in_proj_b, out_proj_w,
           out_proj_b, lin_w, lin_b, ln_w, ln_b):
    emb_pad = jnp.zeros((N_PAD, D), jnp.float32).at[:N].set(embeddings)
    src = edge_index[0].astype(jnp.int32)
    dst = edge_index[1].astype(jnp.int32)

    mask_ref = jax.new_ref(jnp.zeros((FLAT,), jnp.float32))
    _sc_scatter(src, dst, mask_ref)
    mask2d = mask_ref[...].reshape(N_PAD, N_PAD)

    q, k, v = _proj(emb_pad, in_proj_w, in_proj_b.reshape(1, 3 * D))
    out = _flash(q, k, v, mask2d, emb_pad,
                 out_proj_w, out_proj_b.reshape(1, D),
                 lin_w, lin_b.reshape(1, D),
                 ln_w.reshape(1, D), ln_b.reshape(1, D))
    return out[:N]


# speedup vs baseline: 19.4665x; 1.0939x over previous
import functools

import jax
import jax.numpy as jnp
from jax import lax
from jax.experimental import pallas as pl
from jax.experimental.pallas import tpu as pltpu
from jax.experimental.pallas import tpu_sc as plsc

N = 10000
NE = 320000
D = 128
H = 8
DH = D // H
SCALE = 1.0 / (DH ** 0.5)

N_PAD = 10240
FLAT = N_PAD * N_PAD
BC = 256
BN = 512
NI = N_PAD // BC
NJ = N_PAD // BN

NC = 2
NS = 16
NW = NC * NS
E_PER_W = NE // NW
EBATCH = 2000
NBATCH = E_PER_W // EBATCH
VPB = EBATCH // 16
CHUNKS = 160


def _sc_scatter_body(src_hbm, dst_hbm, mask_hbm, idx2d, val2d, sbuf, dbuf,
                     ssem):
    cid = lax.axis_index("c")
    sid = lax.axis_index("s")
    wid = cid * NS + sid
    dumpbase = N

    lanes = lax.iota(jnp.int32, 16)
    for r in range(156, CHUNKS):
        for g in range(8):
            idx2d[r, pl.ds(g * 16, 16)] = dumpbase + g * 16 + lanes
            val2d[r, pl.ds(g * 16, 16)] = jnp.zeros((16,), jnp.float32)

    ebase = wid * E_PER_W

    def batch_body(b, c):
        pltpu.sync_copy(src_hbm.at[pl.ds(ebase + b * EBATCH, EBATCH)], sbuf)
        pltpu.sync_copy(dst_hbm.at[pl.ds(ebase + b * EBATCH, EBATCH)], dbuf)

        def vec_body(i, cc):
            e = b * VPB + i
            r = e // 4
            col = (e % 4) * 32
            s16 = sbuf[pl.ds(i * 16, 16)]
            d16 = dbuf[pl.ds(i * 16, 16)]
            nsl = s16 != d16
            f1 = s16 * N_PAD + d16
            f2 = d16 * N_PAD + s16
            one = jnp.ones((16,), jnp.float32)
            zero = jnp.zeros((16,), jnp.float32)
            idx2d[r, pl.ds(col, 16)] = jnp.where(nsl, f1, dumpbase + col + lanes)
            val2d[r, pl.ds(col, 16)] = jnp.where(nsl, one, zero)
            idx2d[r, pl.ds(col + 16, 16)] = jnp.where(
                nsl, f2, dumpbase + col + 16 + lanes)
            val2d[r, pl.ds(col + 16, 16)] = jnp.where(nsl, one, zero)
            return cc
        lax.fori_loop(0, VPB, vec_body, 0)
        return c
    lax.fori_loop(0, NBATCH, batch_body, 0)

    def sc_outer(t, c):
        descs = []
        for kk in range(32):
            descs.append(pltpu.async_copy(
                val2d.at[t * 32 + kk], mask_hbm.at[idx2d.at[t * 32 + kk]],
                ssem))
        for dsc in descs:
            dsc.wait()
        return c
    lax.fori_loop(0, CHUNKS // 32, sc_outer, 0)


_sc_scatter = pl.kernel(
    _sc_scatter_body,
    out_type=(),
    mesh=plsc.VectorSubcoreMesh(core_axis_name="c", subcore_axis_name="s"),
    scratch_types=[
        pltpu.VMEM((CHUNKS, 128), jnp.int32),
        pltpu.VMEM((CHUNKS, 128), jnp.float32),
        pltpu.VMEM((EBATCH,), jnp.int32),
        pltpu.VMEM((EBATCH,), jnp.int32),
        pltpu.SemaphoreType.DMA,
    ],
)


VW = 32


def _proj_body(emb_ref, w_ref, b_ref, q_ref, k_ref, v_ref):
    bp = emb_ref.shape[0]
    y = lax.dot_general(emb_ref[...], w_ref[...], (((1,), (1,)), ((), ())),
                        preferred_element_type=jnp.float32) + b_ref[...]
    q_ref[...] = y[:, :D] * SCALE
    k_ref[...] = y[:, D:2 * D]
    col = lax.broadcasted_iota(jnp.int32, (bp, VW - DH), 1)
    tail = jnp.where(col == 0, 1.0, 0.0)
    for h in range(H):
        v_ref[:, VW * h:VW * h + DH] = y[:, 2 * D + DH * h:2 * D + DH * (h + 1)]
        v_ref[:, VW * h + DH:VW * (h + 1)] = tail


def _proj(emb_pad, w_all, b_all):
    bp = 512
    return pl.pallas_call(
        _proj_body,
        grid=(N_PAD // bp,),
        in_specs=[
            pl.BlockSpec((bp, D), lambda i: (i, 0)),
            pl.BlockSpec((3 * D, D), lambda i: (0, 0)),
            pl.BlockSpec((1, 3 * D), lambda i: (0, 0)),
        ],
        out_specs=[
            pl.BlockSpec((bp, D), lambda i: (i, 0)),
            pl.BlockSpec((bp, D), lambda i: (i, 0)),
            pl.BlockSpec((bp, H * VW), lambda i: (i, 0)),
        ],
        out_shape=[jax.ShapeDtypeStruct((N_PAD, D), jnp.float32),
                   jax.ShapeDtypeStruct((N_PAD, D), jnp.float32),
                   jax.ShapeDtypeStruct((N_PAD, H * VW), jnp.float32)],
    )(emb_pad, w_all, b_all)


def _flash_body(q_ref, k_ref, v_ref, mask_ref, emb_ref, wo_ref, bo_ref,
                wl_ref, bl_ref, lnw_ref, lnb_ref, out_ref,
                acc, mscr, lscr):
    j = pl.program_id(1)

    @pl.when(j == 0)
    def _():
        acc[...] = jnp.zeros((BC, D), jnp.float32)
        mscr[...] = jnp.full((BC, H), -jnp.inf, jnp.float32)
        lscr[...] = jnp.zeros((BC, H), jnp.float32)

    neg = -jnp.inf
    bias = jnp.where(mask_ref[...] > 0.0, 0.0, neg)
    for h in range(H):
        qh = q_ref[:, h * DH:(h + 1) * DH]
        kh = k_ref[pl.ds(j * BN, BN), h * DH:(h + 1) * DH]
        s = lax.dot_general(qh, kh, (((1,), (1,)), ((), ())),
                            preferred_element_type=jnp.float32) + bias
        mo = mscr[:, h:h + 1]
        mn = jnp.maximum(mo, jnp.max(s, axis=1, keepdims=True))
        msafe = jnp.where(mn > neg, mn, 0.0)
        p = jnp.exp(s - msafe)
        alpha = jnp.where(mn > neg, jnp.exp(mo - mn), 0.0)
        vh = v_ref[pl.ds(j * BN, BN), VW * h:VW * (h + 1)]
        pv = lax.dot_general(p, vh, (((1,), (0,)), ((), ())),
                             preferred_element_type=jnp.float32)
        acc[:, h * DH:(h + 1) * DH] = acc[:, h * DH:(h + 1) * DH] * alpha + \
            pv[:, :DH]
        lscr[:, h:h + 1] = lscr[:, h:h + 1] * alpha + pv[:, DH:DH + 1]
        mscr[:, h:h + 1] = mn

    @pl.when(j == NJ - 1)
    def _():
        parts = [acc[:, h * DH:(h + 1) * DH] / lscr[:, h:h + 1]
                 for h in range(H)]
        ctx = jnp.concatenate(parts, axis=1)
        ctxp = lax.dot_general(ctx, wo_ref[...], (((1,), (1,)), ((), ())),
                               preferred_element_type=jnp.float32) + bo_ref[...]
        has = lscr[:, 0:1] > 0.0
        c2 = jnp.where(has, ctxp, emb_ref[...])
        h1 = lax.dot_general(c2, wl_ref[...], (((1,), (1,)), ((), ())),
                             preferred_element_type=jnp.float32) + bl_ref[...]
        mu = jnp.mean(h1, axis=1, keepdims=True)
        var = jnp.mean((h1 - mu) ** 2, axis=1, keepdims=True)
        hn = (h1 - mu) / jnp.sqrt(var + 1e-5) * lnw_ref[...] + lnb_ref[...]
        out_ref[...] = 0.5 * hn * (1.0 + lax.erf(hn * (2.0 ** -0.5)))


def _flash(q, k, v, mask2d, emb_pad, wo, bo, wl, bl, lnw, lnb):
    return pl.pallas_call(
        _flash_body,
        grid=(NI, NJ),
        in_specs=[
            pl.BlockSpec((BC, D), lambda i, j: (i, 0)),
            pl.BlockSpec((N_PAD, D), lambda i, j: (0, 0)),
            pl.BlockSpec((N_PAD, H * VW), lambda i, j: (0, 0)),
            pl.BlockSpec((BC, BN), lambda i, j: (i, j)),
            pl.BlockSpec((BC, D), lambda i, j: (i, 0)),
            pl.BlockSpec((D, D), lambda i, j: (0, 0)),
            pl.BlockSpec((1, D), lambda i, j: (0, 0)),
            pl.BlockSpec((D, D), lambda i, j: (0, 0)),
            pl.BlockSpec((1, D), lambda i, j: (0, 0)),
            pl.BlockSpec((1, D), lambda i, j: (0, 0)),
            pl.BlockSpec((1, D), lambda i, j: (0, 0)),
        ],
        out_specs=pl.BlockSpec((BC, D), lambda i, j: (i, 0)),
        out_shape=jax.ShapeDtypeStruct((N_PAD, D), jnp.float32),
        scratch_shapes=[
            pltpu.VMEM((BC, D), jnp.float32),
            pltpu.VMEM((BC, H), jnp.float32),
            pltpu.VMEM((BC, H), jnp.float32),
        ],
        compiler_params=pltpu.CompilerParams(
            dimension_semantics=("arbitrary", "arbitrary")),
    )(q, k, v, mask2d, emb_pad, wo, bo, wl, bl, lnw, lnb)


def kernel(embeddings, edge_index, in_proj_w, in_proj_b, out_proj_w,
           out_proj_b, lin_w, lin_b, ln_w, ln_b):
    emb_pad = jnp.zeros((N_PAD, D), jnp.float32).at[:N].set(embeddings)
    src = edge_index[0].astype(jnp.int32)
    dst = edge_index[1].astype(jnp.int32)

    mask_ref = jax.new_ref(jnp.zeros((FLAT,), jnp.float32))
    _sc_scatter(src, dst, mask_ref)
    mask2d = mask_ref[...].reshape(N_PAD, N_PAD)

    q, k, v = _proj(emb_pad, in_proj_w, in_proj_b.reshape(1, 3 * D))
    out = _flash(q, k, v, mask2d, emb_pad,
                 out_proj_w, out_proj_b.reshape(1, D),
                 lin_w, lin_b.reshape(1, D),
                 ln_w.reshape(1, D), ln_b.reshape(1, D))
    return out[:N]


# speedup vs baseline: 25.2236x; 1.2957x over previous
import functools

import jax
import jax.numpy as jnp
from jax import lax
from jax.experimental import pallas as pl
from jax.experimental.pallas import tpu as pltpu
from jax.experimental.pallas import tpu_sc as plsc

N = 10000
NE = 320000
D = 128
H = 8
DH = D // H
SCALE = 1.0 / (DH ** 0.5)

N_PAD = 10240
FLAT = N_PAD * N_PAD
BC = 256
BN = 512
NI = N_PAD // BC
NJ = N_PAD // BN

NC = 2
NS = 16
NW = NC * NS
E_PER_W = NE // NW
EBATCH = 2000
NBATCH = E_PER_W // EBATCH
VPB = EBATCH // 16
CHUNKS = 160

NPIECE = 8
ROWS_P = N_PAD // NPIECE
NI_P = ROWS_P // BC


def _make_sc_body(lo):
    hi = lo + ROWS_P
    dump = ROWS_P * N_PAD

    def body(src_hbm, dst_hbm, mask_hbm, idx1d, idx2d, ones_v, sbuf, dbuf,
             ssem):
        cid = lax.axis_index("c")
        sid = lax.axis_index("s")
        wid = cid * NS + sid
        lanes = lax.iota(jnp.int32, 16)
        for g in range(8):
            ones_v[pl.ds(g * 16, 16)] = jnp.ones((16,), jnp.float32)

        ebase = wid * E_PER_W

        def batch_body(b, ptr):
            pltpu.sync_copy(src_hbm.at[pl.ds(ebase + b * EBATCH, EBATCH)],
                            sbuf)
            pltpu.sync_copy(dst_hbm.at[pl.ds(ebase + b * EBATCH, EBATCH)],
                            dbuf)

            def vec(i, ptr):
                s16 = sbuf[pl.ds(i * 16, 16)]
                d16 = dbuf[pl.ds(i * 16, 16)]
                nsl = s16 != d16
                in1 = nsl & (s16 >= lo) & (s16 < hi)
                in2 = nsl & (d16 >= lo) & (d16 < hi)
                f1 = (s16 - lo) * N_PAD + d16
                f2 = (d16 - lo) * N_PAD + s16
                cs1 = plsc.cumsum(in1.astype(jnp.int32))
                plsc.store_scatter(idx1d, [ptr + cs1 - 1], f1, mask=in1)
                ptr = ptr + jnp.sum(in1.astype(jnp.int32))
                cs2 = plsc.cumsum(in2.astype(jnp.int32))
                plsc.store_scatter(idx1d, [ptr + cs2 - 1], f2, mask=in2)
                ptr = ptr + jnp.sum(in2.astype(jnp.int32))
                return ptr
            return lax.fori_loop(0, VPB, vec, ptr)
        ptr = lax.fori_loop(0, NBATCH, batch_body, jnp.int32(0))

        def pad(t, c):
            idx1d[pl.ds(ptr + t * 16, 16)] = dump + lanes
            return c
        lax.fori_loop(0, 8, pad, 0)
        nchunks = (ptr + 127) // 128

        def cp(i, c):
            idx2d[i // 8, pl.ds((i % 8) * 16, 16)] = idx1d[pl.ds(i * 16, 16)]
            return c
        lax.fori_loop(0, nchunks * 8, cp, 0)

        def fire(t, c):
            pltpu.async_copy(ones_v, mask_hbm.at[idx2d.at[t]], ssem)
            return c
        lax.fori_loop(0, nchunks, fire, 0)

        def drain(t, c):
            pltpu.make_async_copy(ones_v, mask_hbm.at[idx2d.at[0]],
                                  ssem).wait()
            return c
        lax.fori_loop(0, nchunks, drain, 0)
    return body


def _make_sc_scatter(lo):
    return pl.kernel(
        _make_sc_body(lo),
        out_type=(),
        compiler_params=pltpu.CompilerParams(needs_layout_passes=False),
        mesh=plsc.VectorSubcoreMesh(core_axis_name="c", subcore_axis_name="s"),
        scratch_types=[
            pltpu.VMEM((20608,), jnp.int32),
            pltpu.VMEM((CHUNKS, 128), jnp.int32),
            pltpu.VMEM((128,), jnp.float32),
            pltpu.VMEM((EBATCH,), jnp.int32),
            pltpu.VMEM((EBATCH,), jnp.int32),
            pltpu.SemaphoreType.DMA,
        ],
    )


_SC_SCATTERS = [_make_sc_scatter(p * ROWS_P) for p in range(NPIECE)]


VW = 32


def _proj_body(emb_ref, w_ref, b_ref, q_ref, k_ref, v_ref):
    bp = emb_ref.shape[0]
    y = lax.dot_general(emb_ref[...], w_ref[...], (((1,), (1,)), ((), ())),
                        preferred_element_type=jnp.float32) + b_ref[...]
    q_ref[...] = y[:, :D] * SCALE
    k_ref[...] = y[:, D:2 * D]
    col = lax.broadcasted_iota(jnp.int32, (bp, VW - DH), 1)
    tail = jnp.where(col == 0, 1.0, 0.0)
    for h in range(H):
        v_ref[:, VW * h:VW * h + DH] = y[:, 2 * D + DH * h:2 * D + DH * (h + 1)]
        v_ref[:, VW * h + DH:VW * (h + 1)] = tail


def _proj(emb_pad, w_all, b_all):
    bp = 512
    return pl.pallas_call(
        _proj_body,
        grid=(N_PAD // bp,),
        in_specs=[
            pl.BlockSpec((bp, D), lambda i: (i, 0)),
            pl.BlockSpec((3 * D, D), lambda i: (0, 0)),
            pl.BlockSpec((1, 3 * D), lambda i: (0, 0)),
        ],
        out_specs=[
            pl.BlockSpec((bp, D), lambda i: (i, 0)),
            pl.BlockSpec((bp, D), lambda i: (i, 0)),
            pl.BlockSpec((bp, H * VW), lambda i: (i, 0)),
        ],
        out_shape=[jax.ShapeDtypeStruct((N_PAD, D), jnp.float32),
                   jax.ShapeDtypeStruct((N_PAD, D), jnp.float32),
                   jax.ShapeDtypeStruct((N_PAD, H * VW), jnp.float32)],
    )(emb_pad, w_all, b_all)


def _flash_body(q_ref, k_ref, v_ref, mask_ref, emb_ref, wo_ref, bo_ref,
                wl_ref, bl_ref, lnw_ref, lnb_ref, out_ref,
                acc, mscr, lscr):
    j = pl.program_id(1)

    @pl.when(j == 0)
    def _():
        acc[...] = jnp.zeros((BC, D), jnp.float32)
        mscr[...] = jnp.full((BC, H), -jnp.inf, jnp.float32)
        lscr[...] = jnp.zeros((BC, H), jnp.float32)

    neg = -jnp.inf
    bias = jnp.where(mask_ref[...] > 0.0, 0.0, neg)
    for h in range(H):
        qh = q_ref[:, h * DH:(h + 1) * DH]
        kh = k_ref[pl.ds(j * BN, BN), h * DH:(h + 1) * DH]
        s = lax.dot_general(qh, kh, (((1,), (1,)), ((), ())),
                            preferred_element_type=jnp.float32) + bias
        mo = mscr[:, h:h + 1]
        mn = jnp.maximum(mo, jnp.max(s, axis=1, keepdims=True))
        msafe = jnp.where(mn > neg, mn, 0.0)
        p = jnp.exp(s - msafe)
        alpha = jnp.where(mn > neg, jnp.exp(mo - mn), 0.0)
        vh = v_ref[pl.ds(j * BN, BN), VW * h:VW * (h + 1)]
        pv = lax.dot_general(p, vh, (((1,), (0,)), ((), ())),
                             preferred_element_type=jnp.float32)
        acc[:, h * DH:(h + 1) * DH] = acc[:, h * DH:(h + 1) * DH] * alpha + \
            pv[:, :DH]
        lscr[:, h:h + 1] = lscr[:, h:h + 1] * alpha + pv[:, DH:DH + 1]
        mscr[:, h:h + 1] = mn

    @pl.when(j == NJ - 1)
    def _():
        parts = [acc[:, h * DH:(h + 1) * DH] / lscr[:, h:h + 1]
                 for h in range(H)]
        ctx = jnp.concatenate(parts, axis=1)
        ctxp = lax.dot_general(ctx, wo_ref[...], (((1,), (1,)), ((), ())),
                               preferred_element_type=jnp.float32) + bo_ref[...]
        has = lscr[:, 0:1] > 0.0
        c2 = jnp.where(has, ctxp, emb_ref[...])
        h1 = lax.dot_general(c2, wl_ref[...], (((1,), (1,)), ((), ())),
                             preferred_element_type=jnp.float32) + bl_ref[...]
        mu = jnp.mean(h1, axis=1, keepdims=True)
        var = jnp.mean((h1 - mu) ** 2, axis=1, keepdims=True)
        hn = (h1 - mu) / jnp.sqrt(var + 1e-5) * lnw_ref[...] + lnb_ref[...]
        out_ref[...] = 0.5 * hn * (1.0 + lax.erf(hn * (2.0 ** -0.5)))


def _flash_piece(p, q, k, v, maskp, emb_pad, wo, bo, wl, bl, lnw, lnb):
    i0 = p * NI_P

    def cmap(i, j, i0=i0):
        return (i + i0, 0)

    return pl.pallas_call(
        _flash_body,
        grid=(NI_P, NJ),
        in_specs=[
            pl.BlockSpec((BC, D), cmap),
            pl.BlockSpec((N_PAD, D), lambda i, j: (0, 0)),
            pl.BlockSpec((N_PAD, H * VW), lambda i, j: (0, 0)),
            pl.BlockSpec((BC, BN), lambda i, j: (i, j)),
            pl.BlockSpec((BC, D), cmap),
            pl.BlockSpec((D, D), lambda i, j: (0, 0)),
            pl.BlockSpec((1, D), lambda i, j: (0, 0)),
            pl.BlockSpec((D, D), lambda i, j: (0, 0)),
            pl.BlockSpec((1, D), lambda i, j: (0, 0)),
            pl.BlockSpec((1, D), lambda i, j: (0, 0)),
            pl.BlockSpec((1, D), lambda i, j: (0, 0)),
        ],
        out_specs=pl.BlockSpec((BC, D), lambda i, j: (i, 0)),
        out_shape=jax.ShapeDtypeStruct((ROWS_P, D), jnp.float32),
        scratch_shapes=[
            pltpu.VMEM((BC, D), jnp.float32),
            pltpu.VMEM((BC, H), jnp.float32),
            pltpu.VMEM((BC, H), jnp.float32),
        ],
        compiler_params=pltpu.CompilerParams(
            dimension_semantics=("arbitrary", "arbitrary")),
    )(q, k, v, maskp, emb_pad, wo, bo, wl, bl, lnw, lnb)


def kernel(embeddings, edge_index, in_proj_w, in_proj_b, out_proj_w,
           out_proj_b, lin_w, lin_b, ln_w, ln_b):
    emb_pad = jnp.zeros((N_PAD, D), jnp.float32).at[:N].set(embeddings)
    src = edge_index[0].astype(jnp.int32)
    dst = edge_index[1].astype(jnp.int32)

    q, k, v = _proj(emb_pad, in_proj_w, in_proj_b.reshape(1, 3 * D))

    masks = []
    for p in range(NPIECE):
        mref = jax.new_ref(jnp.zeros(((ROWS_P + 8) * N_PAD,), jnp.float32))
        _SC_SCATTERS[p](src, dst, mref)
        masks.append(mref[...].reshape(ROWS_P + 8, N_PAD)[:ROWS_P])

    outs = []
    for p in range(NPIECE):
        outs.append(_flash_piece(
            p, q, k, v, masks[p], emb_pad,
            out_proj_w, out_proj_b.reshape(1, D),
            lin_w, lin_b.reshape(1, D),
            ln_w.reshape(1, D), ln_b.reshape(1, D)))
    return jnp.concatenate(outs, axis=0)[:N]


# speedup vs baseline: 39.3288x; 1.5592x over previous
import functools

import jax
import jax.numpy as jnp
from jax import lax
from jax.experimental import pallas as pl
from jax.experimental.pallas import tpu as pltpu
from jax.experimental.pallas import tpu_sc as plsc

N = 10000
NE = 320000
D = 128
H = 8
DH = D // H
SCALE = 1.0 / (DH ** 0.5)

N_PAD = 10240
FLAT = N_PAD * N_PAD
BC = 256
BN = 1024
NI = N_PAD // BC
NJ = N_PAD // BN

NC = 2
NS = 16
NW = NC * NS
E_PER_W = NE // NW
EBATCH = 2000
NBATCH = E_PER_W // EBATCH
VPB = EBATCH // 16
CHUNKS = 160

NPIECE = 8
ROWS_P = N_PAD // NPIECE
NI_P = ROWS_P // BC


def _make_sc_body(lo):
    hi = lo + ROWS_P
    dump = ROWS_P * N_PAD

    def body(src_hbm, dst_hbm, mask_hbm, idx1d, idx2d, ones_v, sbuf, dbuf,
             ssem):
        cid = lax.axis_index("c")
        sid = lax.axis_index("s")
        wid = cid * NS + sid
        lanes = lax.iota(jnp.int32, 16)
        for g in range(8):
            ones_v[pl.ds(g * 16, 16)] = jnp.ones((16,), jnp.float32)

        ebase = wid * E_PER_W

        def batch_body(b, ptr):
            pltpu.sync_copy(src_hbm.at[pl.ds(ebase + b * EBATCH, EBATCH)],
                            sbuf)
            pltpu.sync_copy(dst_hbm.at[pl.ds(ebase + b * EBATCH, EBATCH)],
                            dbuf)

            def vec(i, ptr):
                s16 = sbuf[pl.ds(i * 16, 16)]
                d16 = dbuf[pl.ds(i * 16, 16)]
                nsl = s16 != d16
                in1 = nsl & (s16 >= lo) & (s16 < hi)
                in2 = nsl & (d16 >= lo) & (d16 < hi)
                f1 = (s16 - lo) * N_PAD + d16
                f2 = (d16 - lo) * N_PAD + s16
                cs1 = plsc.cumsum(in1.astype(jnp.int32))
                plsc.store_scatter(idx1d, [ptr + cs1 - 1], f1, mask=in1)
                ptr = ptr + jnp.sum(in1.astype(jnp.int32))
                cs2 = plsc.cumsum(in2.astype(jnp.int32))
                plsc.store_scatter(idx1d, [ptr + cs2 - 1], f2, mask=in2)
                ptr = ptr + jnp.sum(in2.astype(jnp.int32))
                return ptr
            return lax.fori_loop(0, VPB, vec, ptr)
        ptr = lax.fori_loop(0, NBATCH, batch_body, jnp.int32(0))

        def pad(t, c):
            idx1d[pl.ds(ptr + t * 16, 16)] = dump + lanes
            return c
        lax.fori_loop(0, 8, pad, 0)
        nchunks = (ptr + 127) // 128

        def cp(i, c):
            idx2d[i // 8, pl.ds((i % 8) * 16, 16)] = idx1d[pl.ds(i * 16, 16)]
            return c
        lax.fori_loop(0, nchunks * 8, cp, 0)

        def fire(t, c):
            pltpu.async_copy(ones_v, mask_hbm.at[idx2d.at[t]], ssem)
            return c
        lax.fori_loop(0, nchunks, fire, 0)

        def drain(t, c):
            pltpu.make_async_copy(ones_v, mask_hbm.at[idx2d.at[0]],
                                  ssem).wait()
            return c
        lax.fori_loop(0, nchunks, drain, 0)
    return body


def _make_sc_scatter(lo):
    return pl.kernel(
        _make_sc_body(lo),
        out_type=(),
        compiler_params=pltpu.CompilerParams(needs_layout_passes=False),
        mesh=plsc.VectorSubcoreMesh(core_axis_name="c", subcore_axis_name="s"),
        scratch_types=[
            pltpu.VMEM((20608,), jnp.int32),
            pltpu.VMEM((CHUNKS, 128), jnp.int32),
            pltpu.VMEM((128,), jnp.float32),
            pltpu.VMEM((EBATCH,), jnp.int32),
            pltpu.VMEM((EBATCH,), jnp.int32),
            pltpu.SemaphoreType.DMA,
        ],
    )


_SC_SCATTERS = [_make_sc_scatter(p * ROWS_P) for p in range(NPIECE)]


VW = 32


def _proj_body(emb_ref, w_ref, b_ref, q_ref, k_ref, v_ref):
    bp = emb_ref.shape[0]
    y = lax.dot_general(emb_ref[...], w_ref[...], (((1,), (1,)), ((), ())),
                        preferred_element_type=jnp.float32) + b_ref[...]
    q_ref[...] = y[:, :D] * SCALE
    k_ref[...] = y[:, D:2 * D]
    col = lax.broadcasted_iota(jnp.int32, (bp, VW - DH), 1)
    tail = jnp.where(col == 0, 1.0, 0.0)
    for h in range(H):
        v_ref[:, VW * h:VW * h + DH] = y[:, 2 * D + DH * h:2 * D + DH * (h + 1)]
        v_ref[:, VW * h + DH:VW * (h + 1)] = tail


def _proj(emb_pad, w_all, b_all):
    bp = 512
    return pl.pallas_call(
        _proj_body,
        grid=(N_PAD // bp,),
        in_specs=[
            pl.BlockSpec((bp, D), lambda i: (i, 0)),
            pl.BlockSpec((3 * D, D), lambda i: (0, 0)),
            pl.BlockSpec((1, 3 * D), lambda i: (0, 0)),
        ],
        out_specs=[
            pl.BlockSpec((bp, D), lambda i: (i, 0)),
            pl.BlockSpec((bp, D), lambda i: (i, 0)),
            pl.BlockSpec((bp, H * VW), lambda i: (i, 0)),
        ],
        out_shape=[jax.ShapeDtypeStruct((N_PAD, D), jnp.float32),
                   jax.ShapeDtypeStruct((N_PAD, D), jnp.float32),
                   jax.ShapeDtypeStruct((N_PAD, H * VW), jnp.float32)],
    )(emb_pad, w_all, b_all)


def _flash_body(q_ref, k_ref, v_ref, mask_ref, emb_ref, wo_ref, bo_ref,
                wl_ref, bl_ref, lnw_ref, lnb_ref, out_ref,
                acc, mscr):
    j = pl.program_id(1)

    @pl.when(j == 0)
    def _():
        acc[...] = jnp.zeros((BC, H * VW), jnp.float32)
        mscr[...] = jnp.full((BC, H), -jnp.inf, jnp.float32)

    neg = -jnp.inf
    bias = jnp.where(mask_ref[...] > 0.0, 0.0, neg)
    for h in range(H):
        qh = q_ref[:, h * DH:(h + 1) * DH]
        kh = k_ref[pl.ds(j * BN, BN), h * DH:(h + 1) * DH]
        s = lax.dot_general(qh, kh, (((1,), (1,)), ((), ())),
                            preferred_element_type=jnp.float32) + bias
        mo = mscr[:, h:h + 1]
        mn = jnp.maximum(mo, jnp.max(s, axis=1, keepdims=True))
        msafe = jnp.where(mn > neg, mn, 0.0)
        p = jnp.exp(s - msafe)
        alpha = jnp.where(mn > neg, jnp.exp(mo - mn), 0.0)
        vh = v_ref[pl.ds(j * BN, BN), VW * h:VW * (h + 1)]
        pv = lax.dot_general(p, vh, (((1,), (0,)), ((), ())),
                             preferred_element_type=jnp.float32)
        acc[:, VW * h:VW * (h + 1)] = acc[:, VW * h:VW * (h + 1)] * alpha + pv
        mscr[:, h:h + 1] = mn

    @pl.when(j == NJ - 1)
    def _():
        parts = [acc[:, VW * h:VW * h + DH] /
                 acc[:, VW * h + DH:VW * h + DH + 1] for h in range(H)]
        ctx = jnp.concatenate(parts, axis=1)
        ctxp = lax.dot_general(ctx, wo_ref[...], (((1,), (1,)), ((), ())),
                               preferred_element_type=jnp.float32) + bo_ref[...]
        has = acc[:, DH:DH + 1] > 0.0
        c2 = jnp.where(has, ctxp, emb_ref[...])
        h1 = lax.dot_general(c2, wl_ref[...], (((1,), (1,)), ((), ())),
                             preferred_element_type=jnp.float32) + bl_ref[...]
        mu = jnp.mean(h1, axis=1, keepdims=True)
        var = jnp.mean((h1 - mu) ** 2, axis=1, keepdims=True)
        hn = (h1 - mu) / jnp.sqrt(var + 1e-5) * lnw_ref[...] + lnb_ref[...]
        out_ref[...] = 0.5 * hn * (1.0 + lax.erf(hn * (2.0 ** -0.5)))


def _flash_piece(p, q, k, v, maskp, emb_pad, wo, bo, wl, bl, lnw, lnb):
    i0 = p * NI_P

    def cmap(i, j, i0=i0):
        return (i + i0, 0)

    return pl.pallas_call(
        _flash_body,
        grid=(NI_P, NJ),
        in_specs=[
            pl.BlockSpec((BC, D), cmap),
            pl.BlockSpec((N_PAD, D), lambda i, j: (0, 0)),
            pl.BlockSpec((N_PAD, H * VW), lambda i, j: (0, 0)),
            pl.BlockSpec((BC, BN), lambda i, j: (i, j)),
            pl.BlockSpec((BC, D), cmap),
            pl.BlockSpec((D, D), lambda i, j: (0, 0)),
            pl.BlockSpec((1, D), lambda i, j: (0, 0)),
            pl.BlockSpec((D, D), lambda i, j: (0, 0)),
            pl.BlockSpec((1, D), lambda i, j: (0, 0)),
            pl.BlockSpec((1, D), lambda i, j: (0, 0)),
            pl.BlockSpec((1, D), lambda i, j: (0, 0)),
        ],
        out_specs=pl.BlockSpec((BC, D), lambda i, j: (i, 0)),
        out_shape=jax.ShapeDtypeStruct((ROWS_P, D), jnp.float32),
        scratch_shapes=[
            pltpu.VMEM((BC, H * VW), jnp.float32),
            pltpu.VMEM((BC, H), jnp.float32),
        ],
        compiler_params=pltpu.CompilerParams(
            dimension_semantics=("arbitrary", "arbitrary")),
    )(q, k, v, maskp, emb_pad, wo, bo, wl, bl, lnw, lnb)


def kernel(embeddings, edge_index, in_proj_w, in_proj_b, out_proj_w,
           out_proj_b, lin_w, lin_b, ln_w, ln_b):
    emb_pad = jnp.zeros((N_PAD, D), jnp.float32).at[:N].set(embeddings)
    src = edge_index[0].astype(jnp.int32)
    dst = edge_index[1].astype(jnp.int32)

    q, k, v = _proj(emb_pad, in_proj_w, in_proj_b.reshape(1, 3 * D))

    masks = []
    for p in range(NPIECE):
        mref = jax.new_ref(jnp.zeros(((ROWS_P + 8) * N_PAD,), jnp.float32))
        _SC_SCATTERS[p](src, dst, mref)
        masks.append(mref[...].reshape(ROWS_P + 8, N_PAD))

    outs = []
    for p in range(NPIECE):
        outs.append(_flash_piece(
            p, q, k, v, masks[p], emb_pad,
            out_proj_w, out_proj_b.reshape(1, D),
            lin_w, lin_b.reshape(1, D),
            ln_w.reshape(1, D), ln_b.reshape(1, D)))
    return jnp.concatenate(outs, axis=0)[:N]
